# baseline (speedup 1.0000x reference)
import functools

import jax
import jax.numpy as jnp
from jax import lax
from jax.experimental import pallas as pl
from jax.experimental.pallas import tpu as pltpu
from jax.experimental.pallas import tpu_sc as plsc

L = 16
ROW = 1024
NB = ROW // L
NC, NS = 2, 16
NW = NC * NS
ROWS_TOTAL = 4 * 8192
ROWS_PER_W = ROWS_TOTAL // NW
RBLK = 16
GRP = 16


def _vsort(v, desc=False):
    if desc:
        return plsc.sort_key_val(v, v, descending=True)[0]
    return lax.sort(v, dimension=0)


def _minmax(a, b, desc):
    if desc:
        return jnp.maximum(a, b), jnp.minimum(a, b)
    return jnp.minimum(a, b), jnp.maximum(a, b)


def _merge_tail(vals, dv0, desc):
    n2 = len(vals)
    dv = dv0
    while dv >= 1:
        out = list(vals)
        for j in range(n2):
            if (j // dv) % 2 == 0:
                out[j], out[j + dv] = _minmax(vals[j], vals[j + dv], desc)
        vals = out
        dv //= 2
    return [_vsort(v, desc) for v in vals]


def _merge_vals(vals, desc):
    n2 = len(vals)
    mv = n2 // 2
    out = [None] * n2
    for j in range(mv):
        out[j], out[mv + j] = _minmax(vals[j], vals[mv + j], desc)
    return _merge_tail(out, mv // 2, desc)


def _sort_group(vals, final_desc):
    n = len(vals)
    vals = [_vsort(v, desc=(j % 2 == 1)) for j, v in enumerate(vals)]
    mb = 1
    while mb < n:
        out = []
        for t in range(n // (2 * mb)):
            d = final_desc if 2 * mb == n else (t % 2 == 1)
            out.extend(_merge_vals(vals[t * 2 * mb:(t + 1) * 2 * mb], d))
        vals = out
        mb *= 2
    return vals


def _merge_mem(ld, st, mv, desc):
    n2 = 2 * mv
    if n2 <= GRP:
        vals = _merge_vals([ld(j) for j in range(n2)], desc)
        for j in range(n2):
            st(j, vals[j])
        return

    for j in range(mv):
        lo, hi = _minmax(ld(j), ld(mv + j), desc)
        st(j, lo)
        st(mv + j, hi)

    dv = mv // 2
    while dv >= GRP:
        for j in range(n2):
            if (j // dv) % 2 == 0:
                lo, hi = _minmax(ld(j), ld(j + dv), desc)
                st(j, lo)
                st(j + dv, hi)
        dv //= 2

    for g in range(n2 // GRP):
        vals = [ld(g * GRP + j) for j in range(GRP)]
        vals = _merge_tail(vals, GRP // 2, desc)
        for j in range(GRP):
            st(g * GRP + j, vals[j])


RING = 4
NCHUNK = ROWS_PER_W // RBLK


def _make_kernel():
    mesh = plsc.VectorSubcoreMesh(core_axis_name="c", subcore_axis_name="s")

    @functools.partial(
        pl.kernel,
        out_type=jax.ShapeDtypeStruct((ROWS_TOTAL, ROW), jnp.float32),
        mesh=mesh,
        scratch_types=(
            [pltpu.VMEM((RBLK, ROW), jnp.float32)] * RING
            + [pltpu.SemaphoreType.DMA] * (2 * RING)
        ),
        compiler_params=pltpu.CompilerParams(needs_layout_passes=False),
    )
    def sort_kernel(x_hbm, out_hbm, *refs):
        bufs = refs[:RING]
        isems = refs[RING:2 * RING]
        osems = refs[2 * RING:]
        wid = lax.axis_index("s") * NC + lax.axis_index("c")
        row0 = wid * ROWS_PER_W

        def in_copy(ci, b):
            pltpu.async_copy(
                x_hbm.at[pl.ds(row0 + ci * RBLK, RBLK)], bufs[b], isems[b])

        def wait_in(b):
            pltpu.make_async_copy(
                x_hbm.at[pl.ds(row0, RBLK)], bufs[b], isems[b]).wait()

        def out_copy(ci, b):
            pltpu.async_copy(
                bufs[b], out_hbm.at[pl.ds(row0 + ci * RBLK, RBLK)], osems[b])

        def wait_out(b):
            pltpu.make_async_copy(
                bufs[b], out_hbm.at[pl.ds(row0, RBLK)], osems[b]).wait()

        def sort_chunk(buf):
            @plsc.parallel_loop(0, RBLK, 1, unroll=2)
            def row_body(r):
                def ld_at(base):
                    def ld(j):
                        return buf[r, pl.ds(base + j * L, L)]
                    return ld

                def st_at(base):
                    def st(j, v):
                        buf[r, pl.ds(base + j * L, L)] = v
                    return st

                def gpair_body(p, c3):
                    for gb in range(2):
                        base = (2 * p + gb) * GRP * L
                        ld = ld_at(base)
                        st = st_at(base)
                        vals = _sort_group([ld(j) for j in range(GRP)],
                                           final_desc=(gb == 1))
                        for j in range(GRP):
                            st(j, vals[j])
                    return c3

                lax.fori_loop(0, NB // GRP // 2, gpair_body, 0)

                _merge_mem(ld_at(0), st_at(0), 16, False)
                _merge_mem(ld_at(512), st_at(512), 16, True)
                _merge_mem(ld_at(0), st_at(0), 32, False)

        in_copy(0, 0)
        in_copy(1, 1)
        out_copy(2, 2)
        out_copy(3, 3)

        def ring_body(p, carry):
            for b in range(RING):
                ci = p * RING + b
                wait_in(b)
                nb = (b + 2) % RING
                wait_out(nb)
                in_copy(jnp.minimum(ci + 2, NCHUNK - 1), nb)
                sort_chunk(bufs[b])
                out_copy(ci, b)
            return carry

        lax.fori_loop(0, NCHUNK // RING, ring_body, 0)

        wait_in(0)
        wait_in(1)
        wait_out(2)
        wait_out(3)

    return sort_kernel


_sort_rows = _make_kernel()


def kernel(x):
    b, s, n = x.shape
    out = _sort_rows(x.reshape(b * s, n))
    return out.reshape(b, s, n)


# speedup vs baseline: 3.4977x; 3.4977x over previous
import functools

import jax
import jax.numpy as jnp
from jax import lax
from jax.experimental import pallas as pl
from jax.experimental.pallas import tpu as pltpu
from jax.experimental.pallas import tpu_sc as plsc

L = 16
ROW = 1024
NB = ROW // L
NC, NS = 2, 16
NW = NC * NS
ROWS_TOTAL = 4 * 8192
ROWS_PER_W = ROWS_TOTAL // NW
RBLK = 16
GRP = 16


def _vsort(v, desc=False):
    if desc:
        return plsc.sort_key_val(v, v, descending=True)[0]
    return lax.sort(v, dimension=0)


def _minmax(a, b, desc):
    if desc:
        return jnp.maximum(a, b), jnp.minimum(a, b)
    return jnp.minimum(a, b), jnp.maximum(a, b)


def _merge_tail(vals, dv0, desc):
    n2 = len(vals)
    dv = dv0
    while dv >= 1:
        out = list(vals)
        for j in range(n2):
            if (j // dv) % 2 == 0:
                out[j], out[j + dv] = _minmax(vals[j], vals[j + dv], desc)
        vals = out
        dv //= 2
    return [_vsort(v, desc) for v in vals]


def _merge_vals(vals, desc):
    n2 = len(vals)
    mv = n2 // 2
    out = [None] * n2
    for j in range(mv):
        out[j], out[mv + j] = _minmax(vals[j], vals[mv + j], desc)
    return _merge_tail(out, mv // 2, desc)


def _sort_group(vals, final_desc):
    n = len(vals)
    vals = [_vsort(v, desc=(j % 2 == 1)) for j, v in enumerate(vals)]
    mb = 1
    while mb < n:
        out = []
        for t in range(n // (2 * mb)):
            d = final_desc if 2 * mb == n else (t % 2 == 1)
            out.extend(_merge_vals(vals[t * 2 * mb:(t + 1) * 2 * mb], d))
        vals = out
        mb *= 2
    return vals


def _merge_mem(ld, st, mv, desc):
    n2 = 2 * mv
    if n2 <= GRP:
        vals = _merge_vals([ld(j) for j in range(n2)], desc)
        for j in range(n2):
            st(j, vals[j])
        return

    if mv == 2 * GRP:
        for j in range(GRP):
            a, b = ld(j), ld(j + GRP)
            c, d = ld(j + 2 * GRP), ld(j + 3 * GRP)
            a, c = _minmax(a, c, desc)
            b, d = _minmax(b, d, desc)
            a, b = _minmax(a, b, desc)
            c, d = _minmax(c, d, desc)
            st(j, a)
            st(j + GRP, b)
            st(j + 2 * GRP, c)
            st(j + 3 * GRP, d)
    else:
        for j in range(mv):
            lo, hi = _minmax(ld(j), ld(mv + j), desc)
            st(j, lo)
            st(mv + j, hi)

        dv = mv // 2
        while dv >= GRP:
            for j in range(n2):
                if (j // dv) % 2 == 0:
                    lo, hi = _minmax(ld(j), ld(j + dv), desc)
                    st(j, lo)
                    st(j + dv, hi)
            dv //= 2

    for g in range(n2 // GRP):
        vals = [ld(g * GRP + j) for j in range(GRP)]
        vals = _merge_tail(vals, GRP // 2, desc)
        for j in range(GRP):
            st(g * GRP + j, vals[j])


RING = 4
NCHUNK = ROWS_PER_W // RBLK


def _make_kernel():
    mesh = plsc.VectorSubcoreMesh(core_axis_name="c", subcore_axis_name="s")

    @functools.partial(
        pl.kernel,
        out_type=jax.ShapeDtypeStruct((ROWS_TOTAL, ROW), jnp.float32),
        mesh=mesh,
        scratch_types=(
            [pltpu.VMEM((RBLK, ROW), jnp.float32)] * RING
            + [pltpu.SemaphoreType.DMA] * (2 * RING)
        ),
        compiler_params=pltpu.CompilerParams(needs_layout_passes=False),
    )
    def sort_kernel(x_hbm, out_hbm, *refs):
        bufs = refs[:RING]
        isems = refs[RING:2 * RING]
        osems = refs[2 * RING:]
        wid = lax.axis_index("s") * NC + lax.axis_index("c")
        row0 = wid * ROWS_PER_W

        def in_copy(ci, b):
            pltpu.async_copy(
                x_hbm.at[pl.ds(row0 + ci * RBLK, RBLK)], bufs[b], isems[b])

        def wait_in(b):
            pltpu.make_async_copy(
                x_hbm.at[pl.ds(row0, RBLK)], bufs[b], isems[b]).wait()

        def out_copy(ci, b):
            pltpu.async_copy(
                bufs[b], out_hbm.at[pl.ds(row0 + ci * RBLK, RBLK)], osems[b])

        def wait_out(b):
            pltpu.make_async_copy(
                bufs[b], out_hbm.at[pl.ds(row0, RBLK)], osems[b]).wait()

        def sort_chunk(buf):
            def row_body(r, carry2):
                def ld_at(base):
                    def ld(j):
                        return buf[r, pl.ds(base + j * L, L)]
                    return ld

                def st_at(base):
                    def st(j, v):
                        buf[r, pl.ds(base + j * L, L)] = v
                    return st

                for g in range(NB // GRP):
                    base = g * GRP * L
                    ld = ld_at(base)
                    st = st_at(base)
                    vals = _sort_group([ld(j) for j in range(GRP)],
                                       final_desc=(g % 2 == 1))
                    for j in range(GRP):
                        st(j, vals[j])

                _merge_mem(ld_at(0), st_at(0), 16, False)
                _merge_mem(ld_at(512), st_at(512), 16, True)
                _merge_mem(ld_at(0), st_at(0), 32, False)
                return carry2

            lax.fori_loop(0, RBLK, row_body, 0)

        in_copy(0, 0)
        in_copy(1, 1)
        out_copy(2, 2)
        out_copy(3, 3)

        def ring_body(p, carry):
            for b in range(RING):
                ci = p * RING + b
                wait_in(b)
                nb = (b + 2) % RING
                wait_out(nb)
                in_copy(jnp.minimum(ci + 2, NCHUNK - 1), nb)
                sort_chunk(bufs[b])
                out_copy(ci, b)
            return carry

        lax.fori_loop(0, NCHUNK // RING, ring_body, 0)

        wait_in(0)
        wait_in(1)
        wait_out(2)
        wait_out(3)

    return sort_kernel


_sort_rows = _make_kernel()


def kernel(x):
    b, s, n = x.shape
    out = _sort_rows(x.reshape(b * s, n))
    return out.reshape(b, s, n)


# speedup vs baseline: 3.6724x; 1.0500x over previous
import functools

import jax
import jax.numpy as jnp
from jax import lax
from jax.experimental import pallas as pl
from jax.experimental.pallas import tpu as pltpu
from jax.experimental.pallas import tpu_sc as plsc

L = 16
ROW = 1024
NB = ROW // L
NC, NS = 2, 16
NW = NC * NS
ROWS_TOTAL = 4 * 8192
ROWS_PER_W = ROWS_TOTAL // NW
RBLK = 16
GRP = 16


def _vsort(v, desc=False):
    if desc:
        return plsc.sort_key_val(v, v, descending=True)[0]
    return lax.sort(v, dimension=0)


def _minmax(a, b, desc):
    if desc:
        return jnp.maximum(a, b), jnp.minimum(a, b)
    return jnp.minimum(a, b), jnp.maximum(a, b)


def _merge_tail(vals, dv0, desc):
    n2 = len(vals)
    dv = dv0
    while dv >= 1:
        out = list(vals)
        for j in range(n2):
            if (j // dv) % 2 == 0:
                out[j], out[j + dv] = _minmax(vals[j], vals[j + dv], desc)
        vals = out
        dv //= 2
    return [_vsort(v, desc) for v in vals]


def _merge_vals(vals, desc):
    n2 = len(vals)
    mv = n2 // 2
    out = [None] * n2
    for j in range(mv):
        out[j], out[mv + j] = _minmax(vals[j], vals[mv + j], desc)
    return _merge_tail(out, mv // 2, desc)


def _sort_group(vals, final_desc):
    n = len(vals)
    vals = [_vsort(v, desc=(j % 2 == 1)) for j, v in enumerate(vals)]
    mb = 1
    while mb < n:
        out = []
        for t in range(n // (2 * mb)):
            d = final_desc if 2 * mb == n else (t % 2 == 1)
            out.extend(_merge_vals(vals[t * 2 * mb:(t + 1) * 2 * mb], d))
        vals = out
        mb *= 2
    return vals


def _merge_mem(ld, st, mv, desc):
    n2 = 2 * mv
    if n2 <= GRP:
        vals = _merge_vals([ld(j) for j in range(n2)], desc)
        for j in range(n2):
            st(j, vals[j])
        return

    if n2 == 2 * GRP:
        los = []
        for j in range(GRP):
            lo, hi = _minmax(ld(j), ld(GRP + j), desc)
            st(GRP + j, hi)
            los.append(lo)
        vals = _merge_tail(los, GRP // 2, desc)
        for j in range(GRP):
            st(j, vals[j])
        vals = _merge_tail([ld(GRP + j) for j in range(GRP)],
                           GRP // 2, desc)
        for j in range(GRP):
            st(GRP + j, vals[j])
        return

    if mv == 2 * GRP:
        keep = []
        for j in range(GRP):
            a, b = ld(j), ld(j + GRP)
            c, d = ld(j + 2 * GRP), ld(j + 3 * GRP)
            a, c = _minmax(a, c, desc)
            b, d = _minmax(b, d, desc)
            a, b = _minmax(a, b, desc)
            c, d = _minmax(c, d, desc)
            st(j + GRP, b)
            st(j + 2 * GRP, c)
            st(j + 3 * GRP, d)
            keep.append(a)
        vals = _merge_tail(keep, GRP // 2, desc)
        for j in range(GRP):
            st(j, vals[j])
        for g in range(1, 4):
            vals = _merge_tail([ld(g * GRP + j) for j in range(GRP)],
                               GRP // 2, desc)
            for j in range(GRP):
                st(g * GRP + j, vals[j])
        return
    else:
        for j in range(mv):
            lo, hi = _minmax(ld(j), ld(mv + j), desc)
            st(j, lo)
            st(mv + j, hi)

        dv = mv // 2
        while dv >= GRP:
            for j in range(n2):
                if (j // dv) % 2 == 0:
                    lo, hi = _minmax(ld(j), ld(j + dv), desc)
                    st(j, lo)
                    st(j + dv, hi)
            dv //= 2

    for g in range(n2 // GRP):
        vals = [ld(g * GRP + j) for j in range(GRP)]
        vals = _merge_tail(vals, GRP // 2, desc)
        for j in range(GRP):
            st(g * GRP + j, vals[j])


RING = 4
NCHUNK = ROWS_PER_W // RBLK


def _make_kernel():
    mesh = plsc.VectorSubcoreMesh(core_axis_name="c", subcore_axis_name="s")

    @functools.partial(
        pl.kernel,
        out_type=jax.ShapeDtypeStruct((ROWS_TOTAL, ROW), jnp.float32),
        mesh=mesh,
        scratch_types=(
            [pltpu.VMEM((RBLK, ROW), jnp.float32)] * RING
            + [pltpu.SemaphoreType.DMA] * (2 * RING)
        ),
        compiler_params=pltpu.CompilerParams(needs_layout_passes=False),
    )
    def sort_kernel(x_hbm, out_hbm, *refs):
        bufs = refs[:RING]
        isems = refs[RING:2 * RING]
        osems = refs[2 * RING:]
        wid = lax.axis_index("s") * NC + lax.axis_index("c")
        row0 = wid * ROWS_PER_W

        def in_copy(ci, b):
            pltpu.async_copy(
                x_hbm.at[pl.ds(row0 + ci * RBLK, RBLK)], bufs[b], isems[b])

        def wait_in(b):
            pltpu.make_async_copy(
                x_hbm.at[pl.ds(row0, RBLK)], bufs[b], isems[b]).wait()

        def out_copy(ci, b):
            pltpu.async_copy(
                bufs[b], out_hbm.at[pl.ds(row0 + ci * RBLK, RBLK)], osems[b])

        def wait_out(b):
            pltpu.make_async_copy(
                bufs[b], out_hbm.at[pl.ds(row0, RBLK)], osems[b]).wait()

        def sort_chunk(buf):
            def row_body(r, carry2):
                def ld_at(base):
                    def ld(j):
                        return buf[r, pl.ds(base + j * L, L)]
                    return ld

                def st_at(base):
                    def st(j, v):
                        buf[r, pl.ds(base + j * L, L)] = v
                    return st

                for g in range(NB // GRP):
                    base = g * GRP * L
                    ld = ld_at(base)
                    st = st_at(base)
                    vals = _sort_group([ld(j) for j in range(GRP)],
                                       final_desc=(g % 2 == 1))
                    for j in range(GRP):
                        st(j, vals[j])

                _merge_mem(ld_at(0), st_at(0), 16, False)
                _merge_mem(ld_at(512), st_at(512), 16, True)
                _merge_mem(ld_at(0), st_at(0), 32, False)
                return carry2

            lax.fori_loop(0, RBLK, row_body, 0)

        in_copy(0, 0)
        in_copy(1, 1)
        out_copy(2, 2)
        out_copy(3, 3)

        def ring_body(p, carry):
            for b in range(RING):
                ci = p * RING + b
                wait_in(b)
                nb = (b + 2) % RING
                wait_out(nb)
                in_copy(jnp.minimum(ci + 2, NCHUNK - 1), nb)
                sort_chunk(bufs[b])
                out_copy(ci, b)
            return carry

        lax.fori_loop(0, NCHUNK // RING, ring_body, 0)

        wait_in(0)
        wait_in(1)
        wait_out(2)
        wait_out(3)

    return sort_kernel


_sort_rows = _make_kernel()


def kernel(x):
    b, s, n = x.shape
    out = _sort_rows(x.reshape(b * s, n))
    return out.reshape(b, s, n)
